# Initial kernel scaffold; baseline (speedup 1.0000x reference)
import jax
import jax.numpy as jnp
from jax.experimental import pallas as pl


def kernel(nfeat, edge_index, W_neigh, b_neigh):
    raise NotImplementedError("write your pallas kernel here")


# speedup vs baseline: 4.8234x; 4.8234x over previous
import dataclasses
import functools

import jax
import jax.numpy as jnp
from jax import lax
from jax.experimental import pallas as pl
from jax.experimental.pallas import tpu as pltpu
from jax.experimental.pallas import tpu_sc as plsc

NC = 2
NS = 16
NW = NC * NS
L = 16
EB = 64
ZR = 32


def _sc_scatter(nfeat, src, dst, rowidx, zeros_c, np_):
    n, d = nfeat.shape
    e = src.shape[0]
    n_blocks = e // EB
    blocks_per_w = n_blocks // NW
    rows_per_tile = np_ // NS
    mesh = plsc.VectorSubcoreMesh(core_axis_name="c", subcore_axis_name="s")
    cp = pltpu.CompilerParams()
    if "needs_layout_passes" in pltpu.CompilerParams.__dataclass_fields__:
        cp = dataclasses.replace(cp, needs_layout_passes=False)

    @functools.partial(
        pl.kernel,
        compiler_params=cp,
        out_type=(
            jax.ShapeDtypeStruct((NC, np_, d), jnp.float32),
            jax.ShapeDtypeStruct((NC, NS, np_), jnp.float32),
        ),
        mesh=mesh,
        scratch_types=[
            pltpu.VMEM_SHARED((np_, d), jnp.float32),
            pltpu.VMEM((EB,), jnp.int32),
            pltpu.VMEM((EB,), jnp.int32),
            pltpu.VMEM((ZR,), jnp.int32),
            pltpu.VMEM((EB, d), jnp.float32),
            pltpu.VMEM((ZR, d), jnp.float32),
            pltpu.VMEM((np_,), jnp.float32),
        ],
    )
    def k(nfeat_hbm, src_hbm, dst_hbm, rowidx_hbm, zeros_hbm,
          out_acc, out_cnt,
          acc_sh, src_v, dst_v, ridx_v, rows_v, zd_v, cnt_v):
        cid = lax.axis_index("c")
        sid = lax.axis_index("s")
        wid = sid * NC + cid

        zero16 = jnp.zeros((L,), jnp.float32)
        one16 = jnp.ones((L,), jnp.float32)

        pltpu.sync_copy(zeros_hbm, zd_v)

        @pl.loop(0, np_ // L)
        def _(i):
            cnt_v[pl.ds(i * L, L)] = zero16

        row0 = sid * rows_per_tile

        @pl.loop(0, rows_per_tile // ZR)
        def _(i):
            r = row0 + i * ZR
            pltpu.sync_copy(rowidx_hbm.at[pl.ds(r, ZR)], ridx_v)
            pltpu.sync_copy(zd_v, acc_sh.at[ridx_v])

        plsc.subcore_barrier()

        @pl.loop(0, blocks_per_w)
        def _(i):
            base = (wid + i * NW) * EB
            pltpu.sync_copy(src_hbm.at[pl.ds(base, EB)], src_v)
            pltpu.sync_copy(dst_hbm.at[pl.ds(base, EB)], dst_v)
            pltpu.sync_copy(nfeat_hbm.at[src_v], rows_v)
            pltpu.sync_copy(rows_v, acc_sh.at[dst_v], add=True)
            for c in range(EB // L):
                idx = dst_v[pl.ds(c * L, L)]
                plsc.addupdate_scatter(cnt_v, [idx], one16)

        plsc.subcore_barrier()

        pltpu.sync_copy(cnt_v, out_cnt.at[cid, sid])

        @pl.loop(0, rows_per_tile // ZR)
        def _(i):
            r = row0 + i * ZR
            pltpu.sync_copy(rowidx_hbm.at[pl.ds(r, ZR)], ridx_v)
            pltpu.sync_copy(acc_sh.at[ridx_v], zd_v)
            pltpu.sync_copy(zd_v, out_acc.at[cid, pl.ds(r, ZR)])

    return k(nfeat, src, dst, rowidx, zeros_c)


def _tc_finish(p0, p1, cnt, nfeat, wt, b):
    n, d = nfeat.shape
    rb = 1000

    def body(p0_ref, p1_ref, c_ref, nf_ref, w_ref, b_ref, o_ref):
        z = p0_ref[...] + p1_ref[...] + 2.0 * nf_ref[...]
        deg = jnp.sum(c_ref[...], axis=1, keepdims=True) + 1.0
        y = jnp.dot(z, w_ref[...], preferred_element_type=jnp.float32)
        o_ref[...] = y / deg + b_ref[...] * (1.0 + 1.0 / deg)

    feat_spec = pl.BlockSpec((rb, d), lambda i: (i, 0))
    cnt_spec = pl.BlockSpec((rb, NW), lambda i: (i, 0))
    return pl.pallas_call(
        body,
        grid=(n // rb,),
        in_specs=[feat_spec, feat_spec, cnt_spec, feat_spec,
                  pl.BlockSpec((d, d), lambda i: (0, 0)),
                  pl.BlockSpec((1, d), lambda i: (0, 0))],
        out_specs=feat_spec,
        out_shape=jax.ShapeDtypeStruct((n, d), jnp.float32),
    )(p0, p1, cnt, nfeat, wt, b)


def kernel(nfeat, edge_index, W_neigh, b_neigh):
    n, d = nfeat.shape
    e = edge_index.shape[1]
    np_ = ((n + NS * 8 - 1) // (NS * 8)) * (NS * 8) + NS * 8
    ep = ((e + NW * EB - 1) // (NW * EB)) * (NW * EB)
    src = edge_index[0].astype(jnp.int32)
    dst = edge_index[1].astype(jnp.int32)
    src = jnp.concatenate([src, jnp.zeros((ep - e,), jnp.int32)])
    dst = jnp.concatenate([dst, jnp.full((ep - e,), n, jnp.int32)])
    rowidx = jnp.arange(np_, dtype=jnp.int32)
    zeros_c = jnp.zeros((ZR, d), jnp.float32)
    acc, cnt = _sc_scatter(nfeat, src, dst, rowidx, zeros_c, np_)
    cnt_t = cnt.reshape(NW, np_)[:, :n].T
    return _tc_finish(acc[0, :n], acc[1, :n], cnt_t, nfeat,
                      W_neigh.T, b_neigh.reshape(1, -1))


# speedup vs baseline: 5.4850x; 1.1372x over previous
import dataclasses
import functools

import jax
import jax.numpy as jnp
from jax import lax
from jax.experimental import pallas as pl
from jax.experimental.pallas import tpu as pltpu
from jax.experimental.pallas import tpu_sc as plsc

NC = 2
NS = 16
NW = NC * NS
L = 16
EB = 128
ZR = 32


def _sc_scatter(nfeat, edges_b, rowidx, zeros_c, np_):
    n, d = nfeat.shape
    n_blocks = edges_b.shape[0]
    blocks_per_w = n_blocks // NW
    rows_per_tile = np_ // NS
    mesh = plsc.VectorSubcoreMesh(core_axis_name="c", subcore_axis_name="s")
    cp = pltpu.CompilerParams()
    if "needs_layout_passes" in pltpu.CompilerParams.__dataclass_fields__:
        cp = dataclasses.replace(cp, needs_layout_passes=False)

    @functools.partial(
        pl.kernel,
        compiler_params=cp,
        out_type=(
            jax.ShapeDtypeStruct((NC, np_, d), jnp.float32),
            jax.ShapeDtypeStruct((NC, NS, np_), jnp.float32),
        ),
        mesh=mesh,
        scratch_types=[
            pltpu.VMEM_SHARED((np_, d), jnp.float32),
            pltpu.VMEM((2, EB), jnp.int32),
            pltpu.VMEM((ZR,), jnp.int32),
            pltpu.VMEM((EB, d), jnp.float32),
            pltpu.VMEM((ZR, d), jnp.float32),
            pltpu.VMEM((np_,), jnp.float32),
        ],
    )
    def k(nfeat_hbm, edges_hbm, rowidx_hbm, zeros_hbm,
          out_acc, out_cnt,
          acc_sh, sidx_v, ridx_v, rows_v, zd_v, cnt_v):
        cid = lax.axis_index("c")
        sid = lax.axis_index("s")
        wid = sid * NC + cid

        zero16 = jnp.zeros((L,), jnp.float32)
        one16 = jnp.ones((L,), jnp.float32)

        pltpu.sync_copy(zeros_hbm, zd_v)

        @pl.loop(0, np_ // L)
        def _(i):
            cnt_v[pl.ds(i * L, L)] = zero16

        row0 = sid * rows_per_tile

        @pl.loop(0, rows_per_tile // ZR)
        def _(i):
            r = row0 + i * ZR
            pltpu.sync_copy(rowidx_hbm.at[pl.ds(r, ZR)], ridx_v)
            pltpu.sync_copy(zd_v, acc_sh.at[ridx_v])

        plsc.subcore_barrier()

        @pl.loop(0, blocks_per_w)
        def _(i):
            r = wid + i * NW
            pltpu.sync_copy(edges_hbm.at[r], sidx_v)
            pltpu.sync_copy(nfeat_hbm.at[sidx_v.at[0]], rows_v)
            pltpu.sync_copy(rows_v, acc_sh.at[sidx_v.at[1]], add=True)
            for c in range(EB // L):
                idx = sidx_v[1, pl.ds(c * L, L)]
                plsc.addupdate_scatter(cnt_v, [idx], one16)

        plsc.subcore_barrier()

        pltpu.sync_copy(cnt_v, out_cnt.at[cid, sid])

        @pl.loop(0, rows_per_tile // ZR)
        def _(i):
            r = row0 + i * ZR
            pltpu.sync_copy(rowidx_hbm.at[pl.ds(r, ZR)], ridx_v)
            pltpu.sync_copy(acc_sh.at[ridx_v], zd_v)
            pltpu.sync_copy(zd_v, out_acc.at[cid, pl.ds(r, ZR)])

    return k(nfeat, edges_b, rowidx, zeros_c)


def _tc_finish(p0, p1, cnt, nfeat, wt, b):
    n, d = nfeat.shape
    rb = 1000

    def body(p0_ref, p1_ref, c_ref, nf_ref, w_ref, b_ref, o_ref):
        z = p0_ref[...] + p1_ref[...] + 2.0 * nf_ref[...]
        deg = jnp.sum(c_ref[...], axis=1, keepdims=True) + 1.0
        y = jnp.dot(z, w_ref[...], preferred_element_type=jnp.float32)
        o_ref[...] = y / deg + b_ref[...] * (1.0 + 1.0 / deg)

    feat_spec = pl.BlockSpec((rb, d), lambda i: (i, 0))
    cnt_spec = pl.BlockSpec((rb, NW), lambda i: (i, 0))
    return pl.pallas_call(
        body,
        grid=(n // rb,),
        in_specs=[feat_spec, feat_spec, cnt_spec, feat_spec,
                  pl.BlockSpec((d, d), lambda i: (0, 0)),
                  pl.BlockSpec((1, d), lambda i: (0, 0))],
        out_specs=feat_spec,
        out_shape=jax.ShapeDtypeStruct((n, d), jnp.float32),
    )(p0, p1, cnt, nfeat, wt, b)


def kernel(nfeat, edge_index, W_neigh, b_neigh):
    n, d = nfeat.shape
    e = edge_index.shape[1]
    np_ = ((n + NS * 8 - 1) // (NS * 8)) * (NS * 8) + NS * 8
    ep = ((e + NW * EB - 1) // (NW * EB)) * (NW * EB)
    src = edge_index[0].astype(jnp.int32)
    dst = edge_index[1].astype(jnp.int32)
    src = jnp.concatenate([src, jnp.zeros((ep - e,), jnp.int32)])
    dst = jnp.concatenate([dst, jnp.full((ep - e,), n, jnp.int32)])
    edges_b = jnp.stack([src.reshape(-1, EB), dst.reshape(-1, EB)], axis=1)
    rowidx = jnp.arange(np_, dtype=jnp.int32)
    zeros_c = jnp.zeros((ZR, d), jnp.float32)
    acc, cnt = _sc_scatter(nfeat, edges_b, rowidx, zeros_c, np_)
    cnt_t = cnt.reshape(NW, np_)[:, :n].T
    return _tc_finish(acc[0, :n], acc[1, :n], cnt_t, nfeat,
                      W_neigh.T, b_neigh.reshape(1, -1))


# speedup vs baseline: 5.9010x; 1.0758x over previous
import dataclasses
import functools

import jax
import jax.numpy as jnp
from jax import lax
from jax.experimental import pallas as pl
from jax.experimental.pallas import tpu as pltpu
from jax.experimental.pallas import tpu_sc as plsc

NC = 2
NS = 16
NW = NC * NS
L = 16
EB = 64
ZR = 32


def _sc_scatter(nfeat, edges_b, rowidx, zeros_c, np_):
    n, d = nfeat.shape
    n_blocks = edges_b.shape[0]
    blocks_per_w = n_blocks // NW
    pairs = blocks_per_w // 2
    rows_per_tile = np_ // NS
    mesh = plsc.VectorSubcoreMesh(core_axis_name="c", subcore_axis_name="s")
    cp = pltpu.CompilerParams()
    if "needs_layout_passes" in pltpu.CompilerParams.__dataclass_fields__:
        cp = dataclasses.replace(cp, needs_layout_passes=False)

    @functools.partial(
        pl.kernel,
        compiler_params=cp,
        out_type=(
            jax.ShapeDtypeStruct((NC, np_, d), jnp.float32),
            jax.ShapeDtypeStruct((NC, NS, np_), jnp.float32),
        ),
        mesh=mesh,
        scratch_types=[
            pltpu.VMEM_SHARED((np_, d), jnp.float32),
            pltpu.VMEM((2, EB), jnp.int32),
            pltpu.VMEM((2, EB), jnp.int32),
            pltpu.VMEM((ZR,), jnp.int32),
            pltpu.VMEM((EB, d), jnp.float32),
            pltpu.VMEM((EB, d), jnp.float32),
            pltpu.VMEM((ZR, d), jnp.float32),
            pltpu.VMEM((np_,), jnp.float32),
            pltpu.SemaphoreType.DMA,
            pltpu.SemaphoreType.DMA,
            pltpu.SemaphoreType.DMA,
            pltpu.SemaphoreType.DMA,
        ],
    )
    def k(nfeat_hbm, edges_hbm, rowidx_hbm, zeros_hbm,
          out_acc, out_cnt,
          acc_sh, sidx0, sidx1, ridx_v, rows0, rows1, zd_v, cnt_v,
          sem_g0, sem_g1, sem_s0, sem_s1):
        cid = lax.axis_index("c")
        sid = lax.axis_index("s")
        wid = sid * NC + cid

        zero16 = jnp.zeros((L,), jnp.float32)
        one16 = jnp.ones((L,), jnp.float32)

        pltpu.sync_copy(zeros_hbm, zd_v)

        @pl.loop(0, np_ // L)
        def _(i):
            cnt_v[pl.ds(i * L, L)] = zero16

        row0 = sid * rows_per_tile

        @pl.loop(0, rows_per_tile // ZR)
        def _(i):
            r = row0 + i * ZR
            pltpu.sync_copy(rowidx_hbm.at[pl.ds(r, ZR)], ridx_v)
            pltpu.sync_copy(zd_v, acc_sh.at[ridx_v])

        plsc.subcore_barrier()

        def hist(sidx):
            for c in range(EB // L):
                idx = sidx[1, pl.ds(c * L, L)]
                plsc.addupdate_scatter(cnt_v, [idx], one16)

        def load_idx(sidx, j):
            pltpu.sync_copy(edges_hbm.at[wid + j * NW], sidx)

        def start_gather(sidx, rows, sem):
            pltpu.async_copy(nfeat_hbm.at[sidx.at[0]], rows, sem)

        def wait_gather(sidx, rows, sem):
            pltpu.make_async_copy(nfeat_hbm.at[sidx.at[0]], rows, sem).wait()

        def start_scatter(sidx, rows, sem):
            pltpu.async_copy(rows, acc_sh.at[sidx.at[1]], sem, add=True)

        def wait_scatter(sidx, rows, sem):
            pltpu.make_async_copy(rows, acc_sh.at[sidx.at[1]], sem).wait()

        load_idx(sidx0, 0)
        start_gather(sidx0, rows0, sem_g0)
        load_idx(sidx1, 1)
        start_gather(sidx1, rows1, sem_g1)
        wait_gather(sidx0, rows0, sem_g0)
        hist(sidx0)
        start_scatter(sidx0, rows0, sem_s0)
        wait_gather(sidx1, rows1, sem_g1)
        hist(sidx1)
        start_scatter(sidx1, rows1, sem_s1)

        @pl.loop(1, pairs)
        def _(i):
            wait_scatter(sidx0, rows0, sem_s0)
            load_idx(sidx0, 2 * i)
            start_gather(sidx0, rows0, sem_g0)
            wait_scatter(sidx1, rows1, sem_s1)
            load_idx(sidx1, 2 * i + 1)
            start_gather(sidx1, rows1, sem_g1)
            wait_gather(sidx0, rows0, sem_g0)
            hist(sidx0)
            start_scatter(sidx0, rows0, sem_s0)
            wait_gather(sidx1, rows1, sem_g1)
            hist(sidx1)
            start_scatter(sidx1, rows1, sem_s1)

        wait_scatter(sidx0, rows0, sem_s0)
        wait_scatter(sidx1, rows1, sem_s1)

        plsc.subcore_barrier()

        pltpu.sync_copy(cnt_v, out_cnt.at[cid, sid])

        @pl.loop(0, rows_per_tile // ZR)
        def _(i):
            r = row0 + i * ZR
            pltpu.sync_copy(rowidx_hbm.at[pl.ds(r, ZR)], ridx_v)
            pltpu.sync_copy(acc_sh.at[ridx_v], zd_v)
            pltpu.sync_copy(zd_v, out_acc.at[cid, pl.ds(r, ZR)])

    return k(nfeat, edges_b, rowidx, zeros_c)


def _tc_finish(p0, p1, cnt, nfeat, wt, b):
    n, d = nfeat.shape
    rb = 1000

    def body(p0_ref, p1_ref, c_ref, nf_ref, w_ref, b_ref, o_ref):
        z = p0_ref[...] + p1_ref[...] + 2.0 * nf_ref[...]
        deg = jnp.sum(c_ref[...], axis=1, keepdims=True) + 1.0
        y = jnp.dot(z, w_ref[...], preferred_element_type=jnp.float32)
        o_ref[...] = y / deg + b_ref[...] * (1.0 + 1.0 / deg)

    feat_spec = pl.BlockSpec((rb, d), lambda i: (i, 0))
    cnt_spec = pl.BlockSpec((rb, NW), lambda i: (i, 0))
    return pl.pallas_call(
        body,
        grid=(n // rb,),
        in_specs=[feat_spec, feat_spec, cnt_spec, feat_spec,
                  pl.BlockSpec((d, d), lambda i: (0, 0)),
                  pl.BlockSpec((1, d), lambda i: (0, 0))],
        out_specs=feat_spec,
        out_shape=jax.ShapeDtypeStruct((n, d), jnp.float32),
    )(p0, p1, cnt, nfeat, wt, b)


def kernel(nfeat, edge_index, W_neigh, b_neigh):
    n, d = nfeat.shape
    e = edge_index.shape[1]
    np_ = ((n + NS * 8 - 1) // (NS * 8)) * (NS * 8) + NS * 8
    epq = NW * EB * 2
    ep = ((e + epq - 1) // epq) * epq
    src = edge_index[0].astype(jnp.int32)
    dst = edge_index[1].astype(jnp.int32)
    src = jnp.concatenate([src, jnp.zeros((ep - e,), jnp.int32)])
    dst = jnp.concatenate([dst, jnp.full((ep - e,), n, jnp.int32)])
    edges_b = jnp.stack([src.reshape(-1, EB), dst.reshape(-1, EB)], axis=1)
    rowidx = jnp.arange(np_, dtype=jnp.int32)
    zeros_c = jnp.zeros((ZR, d), jnp.float32)
    acc, cnt = _sc_scatter(nfeat, edges_b, rowidx, zeros_c, np_)
    cnt_t = cnt.reshape(NW, np_)[:, :n].T
    return _tc_finish(acc[0, :n], acc[1, :n], cnt_t, nfeat,
                      W_neigh.T, b_neigh.reshape(1, -1))


# speedup vs baseline: 6.1656x; 1.0448x over previous
import dataclasses
import functools

import jax
import jax.numpy as jnp
from jax import lax
from jax.experimental import pallas as pl
from jax.experimental.pallas import tpu as pltpu
from jax.experimental.pallas import tpu_sc as plsc

NC = 2
NS = 16
NW = NC * NS
L = 16
EB = 64
ZR = 32


def _sc_scatter(nfeat, edges_b, rowidx_z, rowidx_c, zeros_c, np_):
    n, d = nfeat.shape
    n_blocks = edges_b.shape[0]
    blocks_per_w = n_blocks // NW
    pairs = blocks_per_w // 2
    rows_per_tile = np_ // NS
    nz = rows_per_tile // ZR
    ncp = rows_per_tile // EB
    cpairs = ncp // 2
    mesh = plsc.VectorSubcoreMesh(core_axis_name="c", subcore_axis_name="s")
    cp = pltpu.CompilerParams()
    if "needs_layout_passes" in pltpu.CompilerParams.__dataclass_fields__:
        cp = dataclasses.replace(cp, needs_layout_passes=False)

    @functools.partial(
        pl.kernel,
        compiler_params=cp,
        out_type=(
            jax.ShapeDtypeStruct((NC, np_, d), jnp.float32),
            jax.ShapeDtypeStruct((NC, NS, np_), jnp.float32),
        ),
        mesh=mesh,
        scratch_types=[
            pltpu.VMEM_SHARED((np_, d), jnp.float32),
            pltpu.VMEM((2, EB), jnp.int32),
            pltpu.VMEM((2, EB), jnp.int32),
            pltpu.VMEM((nz, ZR), jnp.int32),
            pltpu.VMEM((ncp, EB), jnp.int32),
            pltpu.VMEM((EB, d), jnp.float32),
            pltpu.VMEM((EB, d), jnp.float32),
            pltpu.VMEM((ZR, d), jnp.float32),
            pltpu.VMEM((np_,), jnp.float32),
            pltpu.SemaphoreType.DMA,
            pltpu.SemaphoreType.DMA,
            pltpu.SemaphoreType.DMA,
            pltpu.SemaphoreType.DMA,
        ],
    )
    def k(nfeat_hbm, edges_hbm, ridxz_hbm, ridxc_hbm, zeros_hbm,
          out_acc, out_cnt,
          acc_sh, sidx0, sidx1, ridxz_v, ridxc_v, rows0, rows1, zd_v, cnt_v,
          sem_g0, sem_g1, sem_s0, sem_s1):
        cid = lax.axis_index("c")
        sid = lax.axis_index("s")
        wid = sid * NC + cid

        zero16 = jnp.zeros((L,), jnp.float32)
        one16 = jnp.ones((L,), jnp.float32)

        pltpu.sync_copy(zeros_hbm, zd_v)
        pltpu.sync_copy(ridxz_hbm.at[sid], ridxz_v)
        pltpu.sync_copy(ridxc_hbm.at[sid], ridxc_v)

        @pl.loop(0, np_ // L)
        def _(i):
            cnt_v[pl.ds(i * L, L)] = zero16

        row0 = sid * rows_per_tile

        @pl.loop(0, nz)
        def _(i):
            pltpu.async_copy(zd_v, acc_sh.at[ridxz_v.at[i]], sem_s0)

        @pl.loop(0, nz)
        def _(i):
            pltpu.make_async_copy(zd_v, acc_sh.at[ridxz_v.at[0]],
                                  sem_s0).wait()

        plsc.subcore_barrier()

        def hist(sidx):
            for c in range(EB // L):
                idx = sidx[1, pl.ds(c * L, L)]
                plsc.addupdate_scatter(cnt_v, [idx], one16)

        def load_idx(sidx, j):
            pltpu.sync_copy(edges_hbm.at[wid + j * NW], sidx)

        def start_gather(sidx, rows, sem):
            pltpu.async_copy(nfeat_hbm.at[sidx.at[0]], rows, sem)

        def wait_gather(sidx, rows, sem):
            pltpu.make_async_copy(nfeat_hbm.at[sidx.at[0]], rows, sem).wait()

        def start_scatter(sidx, rows, sem):
            pltpu.async_copy(rows, acc_sh.at[sidx.at[1]], sem, add=True)

        def wait_scatter(sidx, rows, sem):
            pltpu.make_async_copy(rows, acc_sh.at[sidx.at[1]], sem).wait()

        load_idx(sidx0, 0)
        start_gather(sidx0, rows0, sem_g0)
        load_idx(sidx1, 1)
        start_gather(sidx1, rows1, sem_g1)
        wait_gather(sidx0, rows0, sem_g0)
        hist(sidx0)
        start_scatter(sidx0, rows0, sem_s0)
        wait_gather(sidx1, rows1, sem_g1)
        hist(sidx1)
        start_scatter(sidx1, rows1, sem_s1)

        @pl.loop(1, pairs)
        def _(i):
            wait_scatter(sidx0, rows0, sem_s0)
            load_idx(sidx0, 2 * i)
            start_gather(sidx0, rows0, sem_g0)
            wait_scatter(sidx1, rows1, sem_s1)
            load_idx(sidx1, 2 * i + 1)
            start_gather(sidx1, rows1, sem_g1)
            wait_gather(sidx0, rows0, sem_g0)
            hist(sidx0)
            start_scatter(sidx0, rows0, sem_s0)
            wait_gather(sidx1, rows1, sem_g1)
            hist(sidx1)
            start_scatter(sidx1, rows1, sem_s1)

        wait_scatter(sidx0, rows0, sem_s0)
        wait_scatter(sidx1, rows1, sem_s1)

        plsc.subcore_barrier()

        pltpu.sync_copy(cnt_v, out_cnt.at[cid, sid])

        def cstart_gather(c, rows, sem):
            pltpu.async_copy(acc_sh.at[ridxc_v.at[c]], rows, sem)

        def cwait_gather(c, rows, sem):
            pltpu.make_async_copy(acc_sh.at[ridxc_v.at[c]], rows, sem).wait()

        def cstart_write(c, rows, sem):
            pltpu.async_copy(rows, out_acc.at[cid, pl.ds(row0 + c * EB, EB)],
                             sem)

        def cwait_write(c, rows, sem):
            pltpu.make_async_copy(rows,
                                  out_acc.at[cid, pl.ds(row0 + c * EB, EB)],
                                  sem).wait()

        cstart_gather(0, rows0, sem_g0)
        cstart_gather(1, rows1, sem_g1)
        cwait_gather(0, rows0, sem_g0)
        cstart_write(0, rows0, sem_s0)
        cwait_gather(1, rows1, sem_g1)
        cstart_write(1, rows1, sem_s1)

        @pl.loop(1, cpairs)
        def _(i):
            cwait_write(2 * i - 2, rows0, sem_s0)
            cstart_gather(2 * i, rows0, sem_g0)
            cwait_write(2 * i - 1, rows1, sem_s1)
            cstart_gather(2 * i + 1, rows1, sem_g1)
            cwait_gather(2 * i, rows0, sem_g0)
            cstart_write(2 * i, rows0, sem_s0)
            cwait_gather(2 * i + 1, rows1, sem_g1)
            cstart_write(2 * i + 1, rows1, sem_s1)

        cwait_write(ncp - 2, rows0, sem_s0)
        cwait_write(ncp - 1, rows1, sem_s1)

    return k(nfeat, edges_b, rowidx_z, rowidx_c, zeros_c)


def _tc_finish(p0, p1, cnt, nfeat, wt, b):
    n, d = nfeat.shape
    rb = 1000

    def body(p0_ref, p1_ref, c_ref, nf_ref, w_ref, b_ref, o_ref):
        z = p0_ref[...] + p1_ref[...] + 2.0 * nf_ref[...]
        deg = jnp.sum(c_ref[...], axis=1, keepdims=True) + 1.0
        y = jnp.dot(z, w_ref[...], preferred_element_type=jnp.float32)
        o_ref[...] = y / deg + b_ref[...] * (1.0 + 1.0 / deg)

    feat_spec = pl.BlockSpec((rb, d), lambda i: (i, 0))
    cnt_spec = pl.BlockSpec((rb, NW), lambda i: (i, 0))
    return pl.pallas_call(
        body,
        grid=(n // rb,),
        in_specs=[feat_spec, feat_spec, cnt_spec, feat_spec,
                  pl.BlockSpec((d, d), lambda i: (0, 0)),
                  pl.BlockSpec((1, d), lambda i: (0, 0))],
        out_specs=feat_spec,
        out_shape=jax.ShapeDtypeStruct((n, d), jnp.float32),
    )(p0, p1, cnt, nfeat, wt, b)


def kernel(nfeat, edge_index, W_neigh, b_neigh):
    n, d = nfeat.shape
    e = edge_index.shape[1]
    np_ = ((n + NS * 8 - 1) // (NS * 8)) * (NS * 8) + NS * 8
    epq = NW * EB * 2
    ep = ((e + epq - 1) // epq) * epq
    src = edge_index[0].astype(jnp.int32)
    dst = edge_index[1].astype(jnp.int32)
    src = jnp.concatenate([src, jnp.zeros((ep - e,), jnp.int32)])
    dst = jnp.concatenate([dst, jnp.full((ep - e,), n, jnp.int32)])
    edges_b = jnp.stack([src.reshape(-1, EB), dst.reshape(-1, EB)], axis=1)
    rowidx_z = jnp.arange(np_, dtype=jnp.int32).reshape(NS, -1, ZR)
    rowidx_c = jnp.arange(np_, dtype=jnp.int32).reshape(NS, -1, EB)
    zeros_c = jnp.zeros((ZR, d), jnp.float32)
    acc, cnt = _sc_scatter(nfeat, edges_b, rowidx_z, rowidx_c, zeros_c, np_)
    cnt_t = cnt.reshape(NW, np_)[:, :n].T
    return _tc_finish(acc[0, :n], acc[1, :n], cnt_t, nfeat,
                      W_neigh.T, b_neigh.reshape(1, -1))
